# baseline (speedup 1.0000x reference)
import functools

import jax
import jax.numpy as jnp
from jax import lax
from jax.experimental import pallas as pl
from jax.experimental.pallas import tpu as pltpu
from jax.experimental.pallas import tpu_sc as plsc

N_NODES_ = 10000
N_EDGES_ = 320000
D_ = 128

NC = 2
NS = 16
NW = NC * NS

E_PER_W = N_EDGES_ // NW
CHUNK = 80
N_CHUNKS = E_PER_W // CHUNK

_GATHER_DNUMS = lax.GatherDimensionNumbers(
    offset_dims=(), collapsed_slice_dims=(0,), start_index_map=(0,))


def _lane_perm(t, idx):
    return lax.gather(
        t, idx[:, None], _GATHER_DNUMS, slice_sizes=(1,),
        mode=lax.GatherScatterMode.PROMISE_IN_BOUNDS)


def _lane_sum(t):
    lane = lax.iota(jnp.int32, 16)
    for m in (8, 4, 2, 1):
        t = t + _lane_perm(t, lane ^ m)
    return t


def _dot_chunk(rows_a, rows_b, tmp_v, out_v, out_base):
    lane = lax.iota(jnp.int32, 16)

    @plsc.parallel_loop(0, CHUNK, 1, unroll=2)
    def _(e):
        parts = []
        for k in range(4):
            aw = rows_a[e, pl.ds(16 * k, 16)]
            bw = rows_b[e, pl.ds(16 * k, 16)]
            a_hi = lax.bitcast_convert_type(aw, jnp.float32)
            b_hi = lax.bitcast_convert_type(bw, jnp.float32)
            a_lo = lax.bitcast_convert_type(aw << 16, jnp.float32)
            b_lo = lax.bitcast_convert_type(bw << 16, jnp.float32)
            parts.append(a_hi * b_hi + a_lo * b_lo)
        s2 = [parts[0] + parts[1], parts[2] + parts[3]]
        t = _lane_sum(s2[0] + s2[1])
        tmp_v[pl.ds(e * 16, 16)] = t

    def compact_body(g, _):
        acc = jnp.zeros((16,), jnp.float32)
        for i in range(16):
            acc = jnp.where(lane == i, tmp_v[pl.ds((g * 16 + i) * 16, 16)],
                            acc)
        out_v[pl.ds(out_base + g * 16, 16)] = acc
        return 0

    lax.fori_loop(0, CHUNK // 16, compact_body, 0)


def _edge_decoder_kernel(x_src_hbm, x_dst_hbm, idx_src_hbm, idx_dst_hbm,
                         out_hbm, ia0, ib0, ia1, ib1,
                         rows_a0, rows_b0, rows_a1, rows_b1,
                         tmp_v, out_v, si0, si1, sa0, sb0, sa1, sb1):
    wid = lax.axis_index("s") * NC + lax.axis_index("c")
    base = wid * E_PER_W

    def issue_idx(c, ia, ib, si):
        off = base + c * CHUNK
        pltpu.async_copy(idx_src_hbm.at[pl.ds(off, CHUNK)], ia, si)
        pltpu.async_copy(idx_dst_hbm.at[pl.ds(off, CHUNK)], ib, si)

    def wait_idx(ia, ib, si):
        pltpu.make_async_copy(idx_src_hbm.at[pl.ds(0, CHUNK)], ia, si).wait()
        pltpu.make_async_copy(idx_dst_hbm.at[pl.ds(0, CHUNK)], ib, si).wait()

    def issue_rows(ia, ib, ra, rb, sa, sb):
        pltpu.async_copy(x_src_hbm.at[ia], ra, sa)
        pltpu.async_copy(x_dst_hbm.at[ib], rb, sb)

    def wait_rows(ia, ib, ra, rb, sa, sb):
        pltpu.make_async_copy(x_src_hbm.at[ia], ra, sa).wait()
        pltpu.make_async_copy(x_dst_hbm.at[ib], rb, sb).wait()

    pltpu.sync_copy(idx_src_hbm.at[pl.ds(base, CHUNK)], ia0)
    pltpu.sync_copy(idx_dst_hbm.at[pl.ds(base, CHUNK)], ib0)
    issue_rows(ia0, ib0, rows_a0, rows_b0, sa0, sb0)
    issue_idx(1, ia1, ib1, si1)

    def pair_body(g, _):
        c0 = 2 * g
        wait_idx(ia1, ib1, si1)
        issue_rows(ia1, ib1, rows_a1, rows_b1, sa1, sb1)
        wait_rows(ia0, ib0, rows_a0, rows_b0, sa0, sb0)
        issue_idx(c0 + 2, ia0, ib0, si0)
        _dot_chunk(rows_a0, rows_b0, tmp_v, out_v, c0 * CHUNK)
        wait_idx(ia0, ib0, si0)
        issue_rows(ia0, ib0, rows_a0, rows_b0, sa0, sb0)
        wait_rows(ia1, ib1, rows_a1, rows_b1, sa1, sb1)

        @pl.when(g < (N_CHUNKS - 3) // 2)
        def _():
            issue_idx(c0 + 3, ia1, ib1, si1)

        _dot_chunk(rows_a1, rows_b1, tmp_v, out_v, (c0 + 1) * CHUNK)
        return 0

    lax.fori_loop(0, (N_CHUNKS - 1) // 2, pair_body, 0)

    wait_rows(ia0, ib0, rows_a0, rows_b0, sa0, sb0)
    _dot_chunk(rows_a0, rows_b0, tmp_v, out_v, (N_CHUNKS - 1) * CHUNK)

    pltpu.sync_copy(out_v, out_hbm.at[pl.ds(base, E_PER_W)])


@jax.jit
def _edge_decoder(x_src, x_dst, idx_src, idx_dst):
    mesh = plsc.VectorSubcoreMesh(core_axis_name="c", subcore_axis_name="s")
    kfn = functools.partial(
        pl.kernel,
        mesh=mesh,
        compiler_params=pltpu.CompilerParams(use_tc_tiling_on_sc=False),
        out_type=jax.ShapeDtypeStruct((N_EDGES_,), jnp.float32),
        scratch_types=[
            pltpu.VMEM((CHUNK,), jnp.int32),
            pltpu.VMEM((CHUNK,), jnp.int32),
            pltpu.VMEM((CHUNK,), jnp.int32),
            pltpu.VMEM((CHUNK,), jnp.int32),
            pltpu.VMEM((CHUNK, D_ // 2), jnp.int32),
            pltpu.VMEM((CHUNK, D_ // 2), jnp.int32),
            pltpu.VMEM((CHUNK, D_ // 2), jnp.int32),
            pltpu.VMEM((CHUNK, D_ // 2), jnp.int32),
            pltpu.VMEM((CHUNK * 16,), jnp.float32),
            pltpu.VMEM((E_PER_W,), jnp.float32),
            pltpu.SemaphoreType.DMA,
            pltpu.SemaphoreType.DMA,
            pltpu.SemaphoreType.DMA,
            pltpu.SemaphoreType.DMA,
            pltpu.SemaphoreType.DMA,
            pltpu.SemaphoreType.DMA,
        ],
    )(_edge_decoder_kernel)
    return kfn(x_src, x_dst, idx_src, idx_dst)


def _pack_table(x):
    xb = x.astype(jnp.bfloat16).reshape(N_NODES_, D_ // 2, 2)
    return jax.lax.bitcast_convert_type(xb, jnp.int32)


def kernel(x_src, x_dst, edge_label_index):
    idx = edge_label_index.astype(jnp.int32)
    return _edge_decoder(_pack_table(x_src), _pack_table(x_dst),
                         idx[0], idx[1])


# speedup vs baseline: 1.1123x; 1.1123x over previous
import functools

import jax
import jax.numpy as jnp
from jax import lax
from jax.experimental import pallas as pl
from jax.experimental.pallas import tpu as pltpu
from jax.experimental.pallas import tpu_sc as plsc

N_NODES_ = 10000
N_EDGES_ = 320000
D_ = 128

NC = 2
NS = 16
NW = NC * NS

E_PER_W = N_EDGES_ // NW
CHUNK = 80
N_CHUNKS = E_PER_W // CHUNK

_GATHER_DNUMS = lax.GatherDimensionNumbers(
    offset_dims=(), collapsed_slice_dims=(0,), start_index_map=(0,))


def _lane_perm(t, idx):
    return lax.gather(
        t, idx[:, None], _GATHER_DNUMS, slice_sizes=(1,),
        mode=lax.GatherScatterMode.PROMISE_IN_BOUNDS)


def _lane_sum(t):
    lane = lax.iota(jnp.int32, 16)
    for m in (8, 4, 2, 1):
        t = t + _lane_perm(t, lane ^ m)
    return t


def _dot_chunk(rows_a, rows_b, tmp_v, out_v, out_base):
    lane = lax.iota(jnp.int32, 16)

    @plsc.parallel_loop(0, CHUNK, 1, unroll=2)
    def _(e):
        parts = []
        for k in range(4):
            aw = rows_a[e, pl.ds(16 * k, 16)]
            bw = rows_b[e, pl.ds(16 * k, 16)]
            a_hi = lax.bitcast_convert_type(aw, jnp.float32)
            b_hi = lax.bitcast_convert_type(bw, jnp.float32)
            a_lo = lax.bitcast_convert_type(aw << 16, jnp.float32)
            b_lo = lax.bitcast_convert_type(bw << 16, jnp.float32)
            parts.append(a_hi * b_hi + a_lo * b_lo)
        s2 = [parts[0] + parts[1], parts[2] + parts[3]]
        t = _lane_sum(s2[0] + s2[1])
        tmp_v[pl.ds(e * 16, 16)] = t

    def compact_body(g, _):
        acc = jnp.zeros((16,), jnp.float32)
        for i in range(16):
            acc = jnp.where(lane == i, tmp_v[pl.ds((g * 16 + i) * 16, 16)],
                            acc)
        out_v[pl.ds(out_base + g * 16, 16)] = acc
        return 0

    lax.fori_loop(0, CHUNK // 16, compact_body, 0)


def _edge_decoder_kernel(x_src_hbm, x_dst_hbm, idx_src_hbm, idx_dst_hbm,
                         out_hbm, ia0, ib0, ia1, ib1,
                         rows_a0, rows_b0, rows_a1, rows_b1,
                         tmp_v, out_v, sh_a, sh_b, si0, si1, sa0, sb0, sa1,
                         sb1):
    sid = lax.axis_index("s")
    wid = sid * NC + lax.axis_index("c")
    base = wid * E_PER_W

    rows_share = N_NODES_ // NS
    pltpu.sync_copy(x_src_hbm.at[pl.ds(sid * rows_share, rows_share)],
                    sh_a.at[pl.ds(sid * rows_share, rows_share)])
    pltpu.sync_copy(x_dst_hbm.at[pl.ds(sid * rows_share, rows_share)],
                    sh_b.at[pl.ds(sid * rows_share, rows_share)])
    plsc.subcore_barrier()

    def issue_idx(c, ia, ib, si):
        off = base + c * CHUNK
        pltpu.async_copy(idx_src_hbm.at[pl.ds(off, CHUNK)], ia, si)
        pltpu.async_copy(idx_dst_hbm.at[pl.ds(off, CHUNK)], ib, si)

    def wait_idx(ia, ib, si):
        pltpu.make_async_copy(idx_src_hbm.at[pl.ds(0, CHUNK)], ia, si).wait()
        pltpu.make_async_copy(idx_dst_hbm.at[pl.ds(0, CHUNK)], ib, si).wait()

    def issue_rows(ia, ib, ra, rb, sa, sb):
        pltpu.async_copy(sh_a.at[ia], ra, sa)
        pltpu.async_copy(sh_b.at[ib], rb, sb)

    def wait_rows(ia, ib, ra, rb, sa, sb):
        pltpu.make_async_copy(sh_a.at[ia], ra, sa).wait()
        pltpu.make_async_copy(sh_b.at[ib], rb, sb).wait()

    pltpu.sync_copy(idx_src_hbm.at[pl.ds(base, CHUNK)], ia0)
    pltpu.sync_copy(idx_dst_hbm.at[pl.ds(base, CHUNK)], ib0)
    issue_rows(ia0, ib0, rows_a0, rows_b0, sa0, sb0)
    issue_idx(1, ia1, ib1, si1)

    def pair_body(g, _):
        c0 = 2 * g
        wait_idx(ia1, ib1, si1)
        issue_rows(ia1, ib1, rows_a1, rows_b1, sa1, sb1)
        wait_rows(ia0, ib0, rows_a0, rows_b0, sa0, sb0)
        issue_idx(c0 + 2, ia0, ib0, si0)
        _dot_chunk(rows_a0, rows_b0, tmp_v, out_v, c0 * CHUNK)
        wait_idx(ia0, ib0, si0)
        issue_rows(ia0, ib0, rows_a0, rows_b0, sa0, sb0)
        wait_rows(ia1, ib1, rows_a1, rows_b1, sa1, sb1)

        @pl.when(g < (N_CHUNKS - 3) // 2)
        def _():
            issue_idx(c0 + 3, ia1, ib1, si1)

        _dot_chunk(rows_a1, rows_b1, tmp_v, out_v, (c0 + 1) * CHUNK)
        return 0

    lax.fori_loop(0, (N_CHUNKS - 1) // 2, pair_body, 0)

    wait_rows(ia0, ib0, rows_a0, rows_b0, sa0, sb0)
    _dot_chunk(rows_a0, rows_b0, tmp_v, out_v, (N_CHUNKS - 1) * CHUNK)

    pltpu.sync_copy(out_v, out_hbm.at[pl.ds(base, E_PER_W)])


@jax.jit
def _edge_decoder(x_src, x_dst, idx_src, idx_dst):
    mesh = plsc.VectorSubcoreMesh(core_axis_name="c", subcore_axis_name="s")
    kfn = functools.partial(
        pl.kernel,
        mesh=mesh,
        compiler_params=pltpu.CompilerParams(use_tc_tiling_on_sc=False),
        out_type=jax.ShapeDtypeStruct((N_EDGES_,), jnp.float32),
        scratch_types=[
            pltpu.VMEM((CHUNK,), jnp.int32),
            pltpu.VMEM((CHUNK,), jnp.int32),
            pltpu.VMEM((CHUNK,), jnp.int32),
            pltpu.VMEM((CHUNK,), jnp.int32),
            pltpu.VMEM((CHUNK, D_ // 2), jnp.int32),
            pltpu.VMEM((CHUNK, D_ // 2), jnp.int32),
            pltpu.VMEM((CHUNK, D_ // 2), jnp.int32),
            pltpu.VMEM((CHUNK, D_ // 2), jnp.int32),
            pltpu.VMEM((CHUNK * 16,), jnp.float32),
            pltpu.VMEM((E_PER_W,), jnp.float32),
            pltpu.VMEM_SHARED((N_NODES_, D_ // 2), jnp.int32),
            pltpu.VMEM_SHARED((N_NODES_, D_ // 2), jnp.int32),
            pltpu.SemaphoreType.DMA,
            pltpu.SemaphoreType.DMA,
            pltpu.SemaphoreType.DMA,
            pltpu.SemaphoreType.DMA,
            pltpu.SemaphoreType.DMA,
            pltpu.SemaphoreType.DMA,
        ],
    )(_edge_decoder_kernel)
    return kfn(x_src, x_dst, idx_src, idx_dst)


def _pack_table(x):
    xb = x.astype(jnp.bfloat16).reshape(N_NODES_, D_ // 2, 2)
    return jax.lax.bitcast_convert_type(xb, jnp.int32)


def kernel(x_src, x_dst, edge_label_index):
    idx = edge_label_index.astype(jnp.int32)
    return _edge_decoder(_pack_table(x_src), _pack_table(x_dst),
                         idx[0], idx[1])
